# baseline (speedup 1.0000x reference)
import jax
import jax.numpy as jnp
from jax import lax
from jax.experimental import pallas as pl

HEADS = 4
BATCH_SIZE = 25
N_GRAPHS = 4
NUM_GRAPHS = BATCH_SIZE * N_GRAPHS
N = 10000
E = 320000
D = 128
OC0 = 64
E_TOT = E + N
NB = 1000


def _tc1_body(x_ref, w_ref, asrc_ref, adst_ref, tbl_ref, ap_ref):
    h = jnp.dot(x_ref[...], w_ref[...], preferred_element_type=jnp.float32)
    tbl_ref[0] = h[:, 0:128]
    tbl_ref[1] = h[:, 128:256]
    a1s, a2s = [], []
    for hd in range(HEADS):
        hh = h[:, hd * OC0:(hd + 1) * OC0]
        a1s.append(jnp.sum(hh * asrc_ref[hd:hd + 1, :], axis=1, keepdims=True))
        a2s.append(jnp.sum(hh * adst_ref[hd:hd + 1, :], axis=1, keepdims=True))
    ap_ref[...] = jnp.concatenate(a1s + a2s, axis=1)


def _tc1(x, W0, a_src0, a_dst0):
    return pl.pallas_call(
        _tc1_body,
        grid=(N // NB,),
        in_specs=[
            pl.BlockSpec((NB, D), lambda i: (i, 0)),
            pl.BlockSpec((D, 256), lambda i: (0, 0)),
            pl.BlockSpec((HEADS, OC0), lambda i: (0, 0)),
            pl.BlockSpec((HEADS, OC0), lambda i: (0, 0)),
        ],
        out_specs=[
            pl.BlockSpec((2, NB, 128), lambda i: (0, i, 0)),
            pl.BlockSpec((NB, 2 * HEADS), lambda i: (i, 0)),
        ],
        out_shape=[
            jax.ShapeDtypeStruct((2, N, 128), jnp.float32),
            jax.ShapeDtypeStruct((N, 2 * HEADS), jnp.float32),
        ],
    )(x, W0, a_src0, a_dst0)


def _spearman_body(a_ref, out_ref):
    a = a_ref[...]
    n = a.shape[0]
    rows = []
    for i in range(n):
        ai = a[i:i + 1, :]
        lt = jnp.sum((a < ai).astype(jnp.float32), axis=0, keepdims=True)
        if i > 0:
            lt = lt + jnp.sum((a[:i, :] == ai).astype(jnp.float32), axis=0,
                              keepdims=True)
        rows.append(lt)
    ranks = jnp.concatenate(rows, axis=0)
    R = ranks - jnp.sum(ranks, axis=0, keepdims=True) * (1.0 / n)
    cov = lax.dot_general(R, R, (((0,), (0,)), ((), ())),
                          preferred_element_type=jnp.float32)
    d = jnp.sqrt(jnp.clip(jnp.sum(R * R, axis=0), 1e-12, None))
    m = d.shape[0]
    denom = d.reshape(m, 1) * d.reshape(1, m)
    rs = jnp.clip(cov / denom, -1.0 + 1e-7, 1.0 - 1e-7)
    out_ref[...] = 1.0 - jnp.abs(rs)


def _spearman_pallas(o1, o2):
    A = jnp.concatenate([o1, o2], axis=1)
    m = A.shape[1]
    return pl.pallas_call(
        _spearman_body,
        out_shape=jax.ShapeDtypeStruct((m, m), jnp.float32),
    )(A)


def _forward(x, edge_index, batch, W0, a_src0, a_dst0, b0,
             W1, a_src1, a_dst1, b1):
    loops = jnp.arange(N, dtype=jnp.int32)
    sr = jnp.concatenate([edge_index[0].astype(jnp.int32), loops])
    dr = jnp.concatenate([edge_index[1].astype(jnp.int32), loops])

    tbl, apack = _tc1(x, W0, a_src0, a_dst0)
    h0 = jnp.concatenate([tbl[0], tbl[1]], axis=1).reshape(N, HEADS, OC0)
    a1r = apack[:, 0:HEADS]
    a2r = apack[:, HEADS:2 * HEADS]

    e0 = jax.nn.leaky_relu(a1r[sr] + a2r[dr], 0.2)
    ex0 = jnp.exp(e0)
    den0 = jax.ops.segment_sum(ex0, dr, num_segments=N)
    agg0 = jax.ops.segment_sum(h0[sr] * ex0[..., None], dr, num_segments=N)
    agg0 = agg0 / (den0[..., None] + 1e-16)
    x1l = jax.nn.relu(agg0.reshape(N, 256) + b0)

    h1 = (x1l @ W1).reshape(N, HEADS, 1)
    a1b = jnp.sum(h1 * a_src1, axis=-1)
    a2b = jnp.sum(h1 * a_dst1, axis=-1)
    e1 = jax.nn.leaky_relu(a1b[sr] + a2b[dr], 0.2)
    ex1 = jnp.exp(e1)
    den1 = jax.ops.segment_sum(ex1, dr, num_segments=N)
    agg1 = jax.ops.segment_sum(h1[sr] * ex1[..., None], dr, num_segments=N)
    agg1 = agg1 / (den1[..., None] + 1e-16)
    x2l = jax.nn.relu(agg1.reshape(N, HEADS) + b1)

    xm = x2l.mean(axis=-1)
    ssum = jax.ops.segment_sum(xm, batch, num_segments=NUM_GRAPHS)
    cnt = jax.ops.segment_sum(jnp.ones_like(xm), batch, num_segments=NUM_GRAPHS)
    pooled = ssum / jnp.maximum(cnt, 1.0)
    return pooled.reshape(BATCH_SIZE, N_GRAPHS).T


def kernel(x1, edge_index1, batch1, x2, edge_index2, batch2,
           W0, a_src0, a_dst0, b0, W1, a_src1, a_dst1, b1):
    o1 = _forward(x1, edge_index1, batch1, W0, a_src0, a_dst0, b0,
                  W1, a_src1, a_dst1, b1)
    o2 = _forward(x2, edge_index2, batch2, W0, a_src0, a_dst0, b0,
                  W1, a_src1, a_dst1, b1)
    return _spearman_pallas(o1, o2)
